# Initial kernel scaffold; baseline (speedup 1.0000x reference)
import jax
import jax.numpy as jnp
from jax.experimental import pallas as pl


def kernel(x):
    raise NotImplementedError("write your pallas kernel here")


# speedup vs baseline: 1.0088x; 1.0088x over previous
import jax
import jax.numpy as jnp
from jax.experimental import pallas as pl
from jax.experimental.pallas import tpu as pltpu

_BLOCK_ROWS = 512


def _copy_scatter_kernel(x_ref, o_ref):
    o_ref[...] = x_ref[...]
    i = pl.program_id(0)
    n = pl.num_programs(0)
    n_cols = o_ref.shape[1]
    col_ids = jax.lax.broadcasted_iota(jnp.int32, (1, n_cols), 1)

    @pl.when(i == 0)
    def _():
        o_ref[0:1, :] = jnp.where(col_ids == n_cols - 2, 1.0, x_ref[0:1, :])

    @pl.when(i == n - 1)
    def _():
        last = o_ref.shape[0] - 1
        o_ref[last : last + 1, :] = jnp.where(
            col_ids == 1, 5.0, x_ref[last : last + 1, :]
        )


@jax.jit
def kernel(x):
    n_rows, n_cols = x.shape
    grid = n_rows // _BLOCK_ROWS
    return pl.pallas_call(
        _copy_scatter_kernel,
        grid=(grid,),
        in_specs=[pl.BlockSpec((_BLOCK_ROWS, n_cols), lambda i: (i, 0))],
        out_specs=pl.BlockSpec((_BLOCK_ROWS, n_cols), lambda i: (i, 0)),
        out_shape=jax.ShapeDtypeStruct(x.shape, x.dtype),
        compiler_params=pltpu.CompilerParams(
            dimension_semantics=("parallel",),
        ),
    )(x)
